# baseline (speedup 1.0000x reference)
import functools

import jax
import jax.numpy as jnp
from jax import lax
from jax.experimental import pallas as pl
from jax.experimental.pallas import tpu as pltpu
from jax.experimental.pallas import tpu_sc as plsc

N_PAD = 10240
E_CHUNK = 128
N_CHUNKS = 1280
HALF = 128
ROWS_PER_TILE = N_PAD // 16

_MESH = plsc.VectorSubcoreMesh(
    core_axis_name="c", subcore_axis_name="s", num_cores=2, num_subcores=16)


def _deg_body(dst_hbm, ones_hbm, deg_hbm, acc, ones_v, idx_v, bounce):
    c = lax.axis_index("c")
    s = lax.axis_index("s")
    w = s * 2 + c
    n_chunks = dst_hbm.shape[0]
    row0 = s * ROWS_PER_TILE
    pltpu.sync_copy(ones_hbm.at[pl.ds(row0, ROWS_PER_TILE)], bounce)
    pltpu.sync_copy(bounce, acc.at[pl.ds(row0, ROWS_PER_TILE)])
    pltpu.sync_copy(ones_hbm.at[pl.ds(0, E_CHUNK)], ones_v)
    plsc.subcore_barrier()

    def body(i, carry):
        j = w + i * 32
        pltpu.sync_copy(dst_hbm.at[j], idx_v.at[0])
        pltpu.sync_copy(ones_v, acc.at[idx_v.at[0]], add=True)
        return carry

    lax.fori_loop(0, (n_chunks - w + 31) // 32, body, 0)
    plsc.subcore_barrier()
    pltpu.sync_copy(acc.at[pl.ds(row0, ROWS_PER_TILE)], bounce)
    pltpu.sync_copy(bounce, deg_hbm.at[c, pl.ds(row0, ROWS_PER_TILE)])


_deg_call = functools.partial(
    pl.kernel,
    out_type=jax.ShapeDtypeStruct((2, N_PAD, 16), jnp.float32),
    mesh=_MESH,
    scratch_types=[
        pltpu.VMEM_SHARED((N_PAD, 16), jnp.float32),
        pltpu.VMEM((E_CHUNK, 16), jnp.float32),
        pltpu.VMEM((1, E_CHUNK), jnp.int32),
        pltpu.VMEM((ROWS_PER_TILE, 16), jnp.float32),
    ],
)(_deg_body)


def _agg_body(src_hbm, dst_hbm, y_hbm, agg_hbm, acc, rows, sidx, didx, gsem):
    c = lax.axis_index("c")
    s = lax.axis_index("s")
    row0 = s * ROWS_PER_TILE

    def init(i, carry):
        r = row0 + i * E_CHUNK
        pltpu.sync_copy(y_hbm.at[pl.ds(c * N_PAD + r, E_CHUNK)], rows)
        pltpu.sync_copy(rows, acc.at[pl.ds(r, E_CHUNK)])
        return carry

    lax.fori_loop(0, ROWS_PER_TILE // E_CHUNK, init, 0)
    plsc.subcore_barrier()

    n_chunks = dst_hbm.shape[0]

    def body(i, carry):
        j = s + i * 16
        pltpu.sync_copy(src_hbm.at[c, j], sidx.at[0])
        pltpu.sync_copy(dst_hbm.at[j], didx.at[0])
        pltpu.async_copy(y_hbm.at[sidx.at[0]], rows, gsem).wait()
        pltpu.sync_copy(rows, acc.at[didx.at[0]], add=True)
        return carry

    lax.fori_loop(0, (n_chunks - s + 15) // 16, body, 0)
    plsc.subcore_barrier()

    def fini(i, carry):
        r = row0 + i * E_CHUNK
        pltpu.sync_copy(acc.at[pl.ds(r, E_CHUNK)], rows)
        pltpu.sync_copy(rows, agg_hbm.at[c, pl.ds(r, E_CHUNK)])
        return carry

    lax.fori_loop(0, ROWS_PER_TILE // E_CHUNK, fini, 0)


_agg_call = functools.partial(
    pl.kernel,
    out_type=jax.ShapeDtypeStruct((2, N_PAD, HALF), jnp.float32),
    mesh=_MESH,
    scratch_types=[
        pltpu.VMEM_SHARED((N_PAD, HALF), jnp.float32),
        pltpu.VMEM((E_CHUNK, HALF), jnp.float32),
        pltpu.VMEM((1, E_CHUNK), jnp.int32),
        pltpu.VMEM((1, E_CHUNK), jnp.int32),
        pltpu.SemaphoreType.DMA,
    ],
)(_agg_body)


_BLK = 1024


def _prep_body(deg_ref, x_ref, dinv_ref, y0_ref):
    deg = deg_ref[0, :, 0:1] + deg_ref[1, :, 0:1] - 1.0
    dinv = lax.rsqrt(deg)
    dinv_ref[...] = dinv
    y = x_ref[...] * dinv
    y0_ref[0] = y[:, :HALF]
    y0_ref[1] = y[:, HALF:]


def _prep_call(degp, xp):
    return pl.pallas_call(
        _prep_body,
        grid=(N_PAD // _BLK,),
        in_specs=[
            pl.BlockSpec((2, _BLK, 16), lambda i: (0, i, 0)),
            pl.BlockSpec((_BLK, 256), lambda i: (i, 0)),
        ],
        out_specs=[
            pl.BlockSpec((_BLK, 1), lambda i: (i, 0)),
            pl.BlockSpec((2, _BLK, HALF), lambda i: (0, i, 0)),
        ],
        out_shape=[
            jax.ShapeDtypeStruct((N_PAD, 1), jnp.float32),
            jax.ShapeDtypeStruct((2, N_PAD, HALF), jnp.float32),
        ],
    )(degp, xp)


def _mid_body(a_ref, dinv_ref, w1_ref, b1_ref, w2_ref, y1_ref):
    dinv = dinv_ref[...]
    z = jnp.concatenate([a_ref[0], a_ref[1]], axis=1) * dinv
    h = jnp.dot(z, w1_ref[...], preferred_element_type=jnp.float32)
    h = jnp.maximum(h + b1_ref[...], 0.0)
    t = jnp.dot(h, w2_ref[...], preferred_element_type=jnp.float32)
    y = t * dinv
    y1_ref[0] = y[:, :HALF]
    y1_ref[1] = y[:, HALF:]


def _mid_call(a0, dinv, w1, b1, w2):
    return pl.pallas_call(
        _mid_body,
        grid=(N_PAD // _BLK,),
        in_specs=[
            pl.BlockSpec((2, _BLK, HALF), lambda i: (0, i, 0)),
            pl.BlockSpec((_BLK, 1), lambda i: (i, 0)),
            pl.BlockSpec((256, 512), lambda i: (0, 0)),
            pl.BlockSpec((1, 512), lambda i: (0, 0)),
            pl.BlockSpec((512, 256), lambda i: (0, 0)),
        ],
        out_specs=pl.BlockSpec((2, _BLK, HALF), lambda i: (0, i, 0)),
        out_shape=jax.ShapeDtypeStruct((2, N_PAD, HALF), jnp.float32),
    )(a0, dinv, w1, b1, w2)


def _fin_body(a_ref, dinv_ref, b2_ref, o_ref):
    o_ref[...] = (jnp.concatenate([a_ref[0], a_ref[1]], axis=1)
                  * dinv_ref[...] + b2_ref[...])


def _fin_call(a1, dinv, b2):
    return pl.pallas_call(
        _fin_body,
        grid=(N_PAD // _BLK,),
        in_specs=[
            pl.BlockSpec((2, _BLK, HALF), lambda i: (0, i, 0)),
            pl.BlockSpec((_BLK, 1), lambda i: (i, 0)),
            pl.BlockSpec((1, 256), lambda i: (0, 0)),
        ],
        out_specs=pl.BlockSpec((_BLK, 256), lambda i: (i, 0)),
        out_shape=jax.ShapeDtypeStruct((N_PAD, 256), jnp.float32),
    )(a1, dinv, b2)


@jax.jit
def kernel(x, edge_index, W1, b1, W2, b2):
    n = x.shape[0]
    e = edge_index.shape[1]
    ei = edge_index.astype(jnp.int32)
    pad = N_CHUNKS * E_CHUNK - e
    pad_idx = jnp.arange(pad, dtype=jnp.int32)
    srcp = jnp.concatenate([ei[0], pad_idx % n])
    dstp = jnp.concatenate([ei[1], n + pad_idx % (N_PAD - n)])
    dst2 = dstp.reshape(N_CHUNKS, E_CHUNK)
    src2 = jnp.stack([srcp, srcp + N_PAD]).reshape(2, N_CHUNKS, E_CHUNK)
    ones16 = jnp.ones((N_PAD, 16), jnp.float32)
    xp = jnp.pad(x, ((0, N_PAD - n), (0, 0)))

    degp = _deg_call(dst2, ones16)
    dinv, y0 = _prep_call(degp, xp)
    a0 = _agg_call(src2, dst2, y0.reshape(2 * N_PAD, HALF))
    y1 = _mid_call(a0, dinv, W1, b1.reshape(1, -1), W2)
    a1 = _agg_call(src2, dst2, y1.reshape(2 * N_PAD, HALF))
    outp = _fin_call(a1, dinv, b2.reshape(1, -1))
    return outp[:n]


# speedup vs baseline: 1.6007x; 1.6007x over previous
import functools

import jax
import jax.numpy as jnp
from jax import lax
from jax.experimental import pallas as pl
from jax.experimental.pallas import tpu as pltpu
from jax.experimental.pallas import tpu_sc as plsc

N_PAD = 10240
E_CHUNK = 128
N_CHUNKS = 1280
HALF = 128
ROWS_PER_TILE = N_PAD // 16

_MESH = plsc.VectorSubcoreMesh(
    core_axis_name="c", subcore_axis_name="s", num_cores=2, num_subcores=16)


def _deg_body(dst_hbm, ones_hbm, deg_hbm, acc, ones_v, idx_v, bounce):
    c = lax.axis_index("c")
    s = lax.axis_index("s")
    w = s * 2 + c
    n_chunks = dst_hbm.shape[0]
    row0 = s * ROWS_PER_TILE
    pltpu.sync_copy(ones_hbm.at[pl.ds(row0, ROWS_PER_TILE)], bounce)
    pltpu.sync_copy(bounce, acc.at[pl.ds(row0, ROWS_PER_TILE)])
    pltpu.sync_copy(ones_hbm.at[pl.ds(0, E_CHUNK)], ones_v)
    plsc.subcore_barrier()

    def body(i, carry):
        j = w + i * 32
        pltpu.sync_copy(dst_hbm.at[j], idx_v.at[0])
        pltpu.sync_copy(ones_v, acc.at[idx_v.at[0]], add=True)
        return carry

    lax.fori_loop(0, (n_chunks - w + 31) // 32, body, 0)
    plsc.subcore_barrier()
    pltpu.sync_copy(acc.at[pl.ds(row0, ROWS_PER_TILE)], bounce)
    pltpu.sync_copy(bounce, deg_hbm.at[c, pl.ds(row0, ROWS_PER_TILE)])


_deg_call = functools.partial(
    pl.kernel,
    out_type=jax.ShapeDtypeStruct((2, N_PAD, 16), jnp.float32),
    mesh=_MESH,
    scratch_types=[
        pltpu.VMEM_SHARED((N_PAD, 16), jnp.float32),
        pltpu.VMEM((E_CHUNK, 16), jnp.float32),
        pltpu.VMEM((1, E_CHUNK), jnp.int32),
        pltpu.VMEM((ROWS_PER_TILE, 16), jnp.float32),
    ],
)(_deg_body)


def _agg_body(src_hbm, dst_hbm, y_hbm, agg_hbm, acc, rows, sidx, didx,
              gsem, ssem, isem_s, isem_d):
    c = lax.axis_index("c")
    s = lax.axis_index("s")
    row0 = s * ROWS_PER_TILE

    def init(i, carry):
        r = row0 + i * E_CHUNK
        pltpu.sync_copy(y_hbm.at[pl.ds(c * N_PAD + r, E_CHUNK)], rows.at[0])
        pltpu.sync_copy(rows.at[0], acc.at[pl.ds(r, E_CHUNK)])
        return carry

    lax.fori_loop(0, ROWS_PER_TILE // E_CHUNK, init, 0)
    plsc.subcore_barrier()

    ncw = N_CHUNKS // 16

    pltpu.async_copy(src_hbm.at[c, s], sidx.at[0], isem_s.at[0])
    pltpu.async_copy(dst_hbm.at[s], didx.at[0], isem_d.at[0])
    pltpu.async_copy(src_hbm.at[c, s + 16], sidx.at[1], isem_s.at[1])
    pltpu.async_copy(dst_hbm.at[s + 16], didx.at[1], isem_d.at[1])
    pltpu.make_async_copy(src_hbm.at[c, s], sidx.at[0], isem_s.at[0]).wait()
    pltpu.async_copy(y_hbm.at[sidx.at[0]], rows.at[0], gsem.at[0])

    def body(i, carry):
        for k in range(4):
            q = 4 * i + k
            j = s + q * 16
            p = k % 2
            pltpu.make_async_copy(
                y_hbm.at[sidx.at[p]], rows.at[p], gsem.at[p]).wait()
            pltpu.make_async_copy(
                dst_hbm.at[s], didx.at[k], isem_d.at[k]).wait()
            pltpu.async_copy(rows.at[p], acc.at[didx.at[k]], ssem.at[p],
                             add=True)

            @pl.when(q >= 1)
            def _():
                pltpu.make_async_copy(
                    rows.at[1 - p], acc.at[didx.at[(k + 3) % 4]],
                    ssem.at[1 - p]).wait()

            @pl.when(q + 2 < ncw)
            def _():
                pltpu.async_copy(src_hbm.at[c, j + 32], sidx.at[p],
                                 isem_s.at[p])
                pltpu.async_copy(dst_hbm.at[j + 32], didx.at[(k + 2) % 4],
                                 isem_d.at[(k + 2) % 4])

            @pl.when(q + 1 < ncw)
            def _():
                pltpu.make_async_copy(
                    src_hbm.at[c, s], sidx.at[1 - p],
                    isem_s.at[1 - p]).wait()
                pltpu.async_copy(y_hbm.at[sidx.at[1 - p]], rows.at[1 - p],
                                 gsem.at[1 - p])
        return carry

    lax.fori_loop(0, ncw // 4, body, 0)
    pltpu.make_async_copy(
        rows.at[1], acc.at[didx.at[3]], ssem.at[1]).wait()
    plsc.subcore_barrier()

    def fini(i, carry):
        r = row0 + i * E_CHUNK
        pltpu.sync_copy(acc.at[pl.ds(r, E_CHUNK)], rows.at[0])
        pltpu.sync_copy(rows.at[0], agg_hbm.at[c, pl.ds(r, E_CHUNK)])
        return carry

    lax.fori_loop(0, ROWS_PER_TILE // E_CHUNK, fini, 0)


_agg_call = functools.partial(
    pl.kernel,
    out_type=jax.ShapeDtypeStruct((2, N_PAD, HALF), jnp.float32),
    mesh=_MESH,
    scratch_types=[
        pltpu.VMEM_SHARED((N_PAD, HALF), jnp.float32),
        pltpu.VMEM((2, E_CHUNK, HALF), jnp.float32),
        pltpu.VMEM((2, E_CHUNK), jnp.int32),
        pltpu.VMEM((4, E_CHUNK), jnp.int32),
        pltpu.SemaphoreType.DMA((2,)),
        pltpu.SemaphoreType.DMA((2,)),
        pltpu.SemaphoreType.DMA((2,)),
        pltpu.SemaphoreType.DMA((4,)),
    ],
)(_agg_body)


_BLK = 1024


def _prep_body(deg_ref, x_ref, dinv_ref, y0_ref):
    deg = deg_ref[0, :, 0:1] + deg_ref[1, :, 0:1] - 1.0
    dinv = lax.rsqrt(deg)
    dinv_ref[...] = dinv
    y = x_ref[...] * dinv
    y0_ref[0] = y[:, :HALF]
    y0_ref[1] = y[:, HALF:]


def _prep_call(degp, xp):
    return pl.pallas_call(
        _prep_body,
        grid=(N_PAD // _BLK,),
        in_specs=[
            pl.BlockSpec((2, _BLK, 16), lambda i: (0, i, 0)),
            pl.BlockSpec((_BLK, 256), lambda i: (i, 0)),
        ],
        out_specs=[
            pl.BlockSpec((_BLK, 1), lambda i: (i, 0)),
            pl.BlockSpec((2, _BLK, HALF), lambda i: (0, i, 0)),
        ],
        out_shape=[
            jax.ShapeDtypeStruct((N_PAD, 1), jnp.float32),
            jax.ShapeDtypeStruct((2, N_PAD, HALF), jnp.float32),
        ],
    )(degp, xp)


def _mid_body(a_ref, dinv_ref, w1_ref, b1_ref, w2_ref, y1_ref):
    dinv = dinv_ref[...]
    z = jnp.concatenate([a_ref[0], a_ref[1]], axis=1) * dinv
    h = jnp.dot(z, w1_ref[...], preferred_element_type=jnp.float32)
    h = jnp.maximum(h + b1_ref[...], 0.0)
    t = jnp.dot(h, w2_ref[...], preferred_element_type=jnp.float32)
    y = t * dinv
    y1_ref[0] = y[:, :HALF]
    y1_ref[1] = y[:, HALF:]


def _mid_call(a0, dinv, w1, b1, w2):
    return pl.pallas_call(
        _mid_body,
        grid=(N_PAD // _BLK,),
        in_specs=[
            pl.BlockSpec((2, _BLK, HALF), lambda i: (0, i, 0)),
            pl.BlockSpec((_BLK, 1), lambda i: (i, 0)),
            pl.BlockSpec((256, 512), lambda i: (0, 0)),
            pl.BlockSpec((1, 512), lambda i: (0, 0)),
            pl.BlockSpec((512, 256), lambda i: (0, 0)),
        ],
        out_specs=pl.BlockSpec((2, _BLK, HALF), lambda i: (0, i, 0)),
        out_shape=jax.ShapeDtypeStruct((2, N_PAD, HALF), jnp.float32),
    )(a0, dinv, w1, b1, w2)


def _fin_body(a_ref, dinv_ref, b2_ref, o_ref):
    o_ref[...] = (jnp.concatenate([a_ref[0], a_ref[1]], axis=1)
                  * dinv_ref[...] + b2_ref[...])


def _fin_call(a1, dinv, b2):
    return pl.pallas_call(
        _fin_body,
        grid=(N_PAD // _BLK,),
        in_specs=[
            pl.BlockSpec((2, _BLK, HALF), lambda i: (0, i, 0)),
            pl.BlockSpec((_BLK, 1), lambda i: (i, 0)),
            pl.BlockSpec((1, 256), lambda i: (0, 0)),
        ],
        out_specs=pl.BlockSpec((_BLK, 256), lambda i: (i, 0)),
        out_shape=jax.ShapeDtypeStruct((N_PAD, 256), jnp.float32),
    )(a1, dinv, b2)


@jax.jit
def kernel(x, edge_index, W1, b1, W2, b2):
    n = x.shape[0]
    e = edge_index.shape[1]
    ei = edge_index.astype(jnp.int32)
    pad = N_CHUNKS * E_CHUNK - e
    pad_idx = jnp.arange(pad, dtype=jnp.int32)
    srcp = jnp.concatenate([ei[0], pad_idx % n])
    dstp = jnp.concatenate([ei[1], n + pad_idx % (N_PAD - n)])
    dst2 = dstp.reshape(N_CHUNKS, E_CHUNK)
    src2 = jnp.stack([srcp, srcp + N_PAD]).reshape(2, N_CHUNKS, E_CHUNK)
    ones16 = jnp.ones((N_PAD, 16), jnp.float32)
    xp = jnp.pad(x, ((0, N_PAD - n), (0, 0)))

    degp = _deg_call(dst2, ones16)
    dinv, y0 = _prep_call(degp, xp)
    a0 = _agg_call(src2, dst2, y0.reshape(2 * N_PAD, HALF))
    y1 = _mid_call(a0, dinv, W1, b1.reshape(1, -1), W2)
    a1 = _agg_call(src2, dst2, y1.reshape(2 * N_PAD, HALF))
    outp = _fin_call(a1, dinv, b2.reshape(1, -1))
    return outp[:n]


# speedup vs baseline: 1.6836x; 1.0518x over previous
import functools

import jax
import jax.numpy as jnp
from jax import lax
from jax.experimental import pallas as pl
from jax.experimental.pallas import tpu as pltpu
from jax.experimental.pallas import tpu_sc as plsc

N_PAD = 10240
E_CHUNK = 128
N_CHUNKS = 1280
HALF = 128
ROWS_PER_TILE = N_PAD // 16

_MESH = plsc.VectorSubcoreMesh(
    core_axis_name="c", subcore_axis_name="s", num_cores=2, num_subcores=16)


def _deg_body(dst_hbm, ones_hbm, deg_hbm, acc, ones_v, idx_v, bounce,
              dsem):
    c = lax.axis_index("c")
    s = lax.axis_index("s")
    w = s * 2 + c
    ncw = N_CHUNKS // 32
    base = w * ncw
    row0 = s * ROWS_PER_TILE
    pltpu.sync_copy(ones_hbm.at[pl.ds(row0, ROWS_PER_TILE)], bounce)
    pltpu.sync_copy(bounce, acc.at[pl.ds(row0, ROWS_PER_TILE)])
    pltpu.sync_copy(ones_hbm.at[pl.ds(0, E_CHUNK)], ones_v)
    pltpu.sync_copy(dst_hbm.at[pl.ds(base, ncw)], idx_v)
    plsc.subcore_barrier()

    def body(i, carry):
        for k in range(8):
            pltpu.async_copy(ones_v, acc.at[idx_v.at[i * 8 + k]], dsem,
                             add=True)
        for k in range(8):
            pltpu.make_async_copy(ones_v, acc.at[idx_v.at[0]], dsem).wait()
        return carry

    lax.fori_loop(0, ncw // 8, body, 0)
    plsc.subcore_barrier()
    pltpu.sync_copy(acc.at[pl.ds(row0, ROWS_PER_TILE)], bounce)
    pltpu.sync_copy(bounce, deg_hbm.at[c, pl.ds(row0, ROWS_PER_TILE)])


_deg_call = functools.partial(
    pl.kernel,
    out_type=jax.ShapeDtypeStruct((2, N_PAD, 16), jnp.float32),
    mesh=_MESH,
    scratch_types=[
        pltpu.VMEM_SHARED((N_PAD, 16), jnp.float32),
        pltpu.VMEM((E_CHUNK, 16), jnp.float32),
        pltpu.VMEM((N_CHUNKS // 32, E_CHUNK), jnp.int32),
        pltpu.VMEM((ROWS_PER_TILE, 16), jnp.float32),
        pltpu.SemaphoreType.DMA,
    ],
)(_deg_body)


def _agg_body(src_hbm, dst_hbm, y_hbm, agg_hbm, acc, rows, sidx, didx,
              gsem, ssem, isem_s, isem_d):
    c = lax.axis_index("c")
    s = lax.axis_index("s")
    row0 = s * ROWS_PER_TILE

    def init(i, carry):
        r = row0 + i * E_CHUNK
        pltpu.sync_copy(y_hbm.at[pl.ds(c * N_PAD + r, E_CHUNK)], rows.at[0])
        pltpu.sync_copy(rows.at[0], acc.at[pl.ds(r, E_CHUNK)])
        return carry

    lax.fori_loop(0, ROWS_PER_TILE // E_CHUNK, init, 0)
    plsc.subcore_barrier()

    ncw = N_CHUNKS // 16

    pltpu.async_copy(src_hbm.at[c, s], sidx.at[0], isem_s.at[0])
    pltpu.async_copy(dst_hbm.at[s], didx.at[0], isem_d.at[0])
    pltpu.async_copy(src_hbm.at[c, s + 16], sidx.at[1], isem_s.at[1])
    pltpu.async_copy(dst_hbm.at[s + 16], didx.at[1], isem_d.at[1])
    pltpu.make_async_copy(src_hbm.at[c, s], sidx.at[0], isem_s.at[0]).wait()
    pltpu.async_copy(y_hbm.at[sidx.at[0]], rows.at[0], gsem.at[0])

    def body(i, carry):
        for k in range(4):
            q = 4 * i + k
            j = s + q * 16
            p = k % 2
            pltpu.make_async_copy(
                y_hbm.at[sidx.at[p]], rows.at[p], gsem.at[p]).wait()
            pltpu.make_async_copy(
                dst_hbm.at[s], didx.at[k], isem_d.at[k]).wait()
            pltpu.async_copy(rows.at[p], acc.at[didx.at[k]], ssem.at[p],
                             add=True)

            @pl.when(q >= 1)
            def _():
                pltpu.make_async_copy(
                    rows.at[1 - p], acc.at[didx.at[(k + 3) % 4]],
                    ssem.at[1 - p]).wait()

            @pl.when(q + 2 < ncw)
            def _():
                pltpu.async_copy(src_hbm.at[c, j + 32], sidx.at[p],
                                 isem_s.at[p])
                pltpu.async_copy(dst_hbm.at[j + 32], didx.at[(k + 2) % 4],
                                 isem_d.at[(k + 2) % 4])

            @pl.when(q + 1 < ncw)
            def _():
                pltpu.make_async_copy(
                    src_hbm.at[c, s], sidx.at[1 - p],
                    isem_s.at[1 - p]).wait()
                pltpu.async_copy(y_hbm.at[sidx.at[1 - p]], rows.at[1 - p],
                                 gsem.at[1 - p])
        return carry

    lax.fori_loop(0, ncw // 4, body, 0)
    pltpu.make_async_copy(
        rows.at[1], acc.at[didx.at[3]], ssem.at[1]).wait()
    plsc.subcore_barrier()

    def fini(i, carry):
        r = row0 + i * E_CHUNK
        pltpu.sync_copy(acc.at[pl.ds(r, E_CHUNK)], rows.at[0])
        pltpu.sync_copy(rows.at[0], agg_hbm.at[c, pl.ds(r, E_CHUNK)])
        return carry

    lax.fori_loop(0, ROWS_PER_TILE // E_CHUNK, fini, 0)


_agg_call = functools.partial(
    pl.kernel,
    out_type=jax.ShapeDtypeStruct((2, N_PAD, HALF), jnp.float32),
    mesh=_MESH,
    scratch_types=[
        pltpu.VMEM_SHARED((N_PAD, HALF), jnp.float32),
        pltpu.VMEM((2, E_CHUNK, HALF), jnp.float32),
        pltpu.VMEM((2, E_CHUNK), jnp.int32),
        pltpu.VMEM((4, E_CHUNK), jnp.int32),
        pltpu.SemaphoreType.DMA((2,)),
        pltpu.SemaphoreType.DMA((2,)),
        pltpu.SemaphoreType.DMA((2,)),
        pltpu.SemaphoreType.DMA((4,)),
    ],
)(_agg_body)


_BLK = 1024


def _prep_body(deg_ref, x_ref, dinv_ref, y0_ref):
    deg = deg_ref[0, :, 0:1] + deg_ref[1, :, 0:1] - 1.0
    dinv = lax.rsqrt(deg)
    dinv_ref[...] = dinv
    y = x_ref[...] * dinv
    y0_ref[0] = y[:, :HALF]
    y0_ref[1] = y[:, HALF:]


def _prep_call(degp, xp):
    return pl.pallas_call(
        _prep_body,
        grid=(N_PAD // _BLK,),
        in_specs=[
            pl.BlockSpec((2, _BLK, 16), lambda i: (0, i, 0)),
            pl.BlockSpec((_BLK, 256), lambda i: (i, 0)),
        ],
        out_specs=[
            pl.BlockSpec((_BLK, 1), lambda i: (i, 0)),
            pl.BlockSpec((2, _BLK, HALF), lambda i: (0, i, 0)),
        ],
        out_shape=[
            jax.ShapeDtypeStruct((N_PAD, 1), jnp.float32),
            jax.ShapeDtypeStruct((2, N_PAD, HALF), jnp.float32),
        ],
    )(degp, xp)


def _mid_body(a_ref, dinv_ref, w1_ref, b1_ref, w2_ref, y1_ref):
    dinv = dinv_ref[...]
    z = jnp.concatenate([a_ref[0], a_ref[1]], axis=1) * dinv
    h = jnp.dot(z, w1_ref[...], preferred_element_type=jnp.float32)
    h = jnp.maximum(h + b1_ref[...], 0.0)
    t = jnp.dot(h, w2_ref[...], preferred_element_type=jnp.float32)
    y = t * dinv
    y1_ref[0] = y[:, :HALF]
    y1_ref[1] = y[:, HALF:]


def _mid_call(a0, dinv, w1, b1, w2):
    return pl.pallas_call(
        _mid_body,
        grid=(N_PAD // _BLK,),
        in_specs=[
            pl.BlockSpec((2, _BLK, HALF), lambda i: (0, i, 0)),
            pl.BlockSpec((_BLK, 1), lambda i: (i, 0)),
            pl.BlockSpec((256, 512), lambda i: (0, 0)),
            pl.BlockSpec((1, 512), lambda i: (0, 0)),
            pl.BlockSpec((512, 256), lambda i: (0, 0)),
        ],
        out_specs=pl.BlockSpec((2, _BLK, HALF), lambda i: (0, i, 0)),
        out_shape=jax.ShapeDtypeStruct((2, N_PAD, HALF), jnp.float32),
    )(a0, dinv, w1, b1, w2)


def _fin_body(a_ref, dinv_ref, b2_ref, o_ref):
    o_ref[...] = (jnp.concatenate([a_ref[0], a_ref[1]], axis=1)
                  * dinv_ref[...] + b2_ref[...])


def _fin_call(a1, dinv, b2):
    return pl.pallas_call(
        _fin_body,
        grid=(N_PAD // _BLK,),
        in_specs=[
            pl.BlockSpec((2, _BLK, HALF), lambda i: (0, i, 0)),
            pl.BlockSpec((_BLK, 1), lambda i: (i, 0)),
            pl.BlockSpec((1, 256), lambda i: (0, 0)),
        ],
        out_specs=pl.BlockSpec((_BLK, 256), lambda i: (i, 0)),
        out_shape=jax.ShapeDtypeStruct((N_PAD, 256), jnp.float32),
    )(a1, dinv, b2)


@jax.jit
def kernel(x, edge_index, W1, b1, W2, b2):
    n = x.shape[0]
    e = edge_index.shape[1]
    ei = edge_index.astype(jnp.int32)
    pad = N_CHUNKS * E_CHUNK - e
    pad_idx = jnp.arange(pad, dtype=jnp.int32)
    srcp = jnp.concatenate([ei[0], pad_idx % n])
    dstp = jnp.concatenate([ei[1], n + pad_idx % (N_PAD - n)])
    dst2 = dstp.reshape(N_CHUNKS, E_CHUNK)
    src2 = jnp.stack([srcp, srcp + N_PAD]).reshape(2, N_CHUNKS, E_CHUNK)
    ones16 = jnp.ones((N_PAD, 16), jnp.float32)
    xp = jnp.pad(x, ((0, N_PAD - n), (0, 0)))

    degp = _deg_call(dst2, ones16)
    dinv, y0 = _prep_call(degp, xp)
    a0 = _agg_call(src2, dst2, y0.reshape(2 * N_PAD, HALF))
    y1 = _mid_call(a0, dinv, W1, b1.reshape(1, -1), W2)
    a1 = _agg_call(src2, dst2, y1.reshape(2 * N_PAD, HALF))
    outp = _fin_call(a1, dinv, b2.reshape(1, -1))
    return outp[:n]
